# baseline (speedup 1.0000x reference)
import functools

import jax
import jax.numpy as jnp
from jax import lax
from jax.experimental import pallas as pl
from jax.experimental.pallas import tpu as pltpu
from jax.experimental.pallas import tpu_sc as plsc

N_LABELS = 1000
N_DEPTHS = 4
BATCH = 16384
D_WORDS = N_LABELS // 4
D_PAD = 256

NUM_CORES = 2
NUM_SUBCORES = 16
NUM_WORKERS = NUM_CORES * NUM_SUBCORES
B_PER_W = BATCH // NUM_WORKERS
CHUNK = 128
N_CHUNKS = B_PER_W // CHUNK
TOTAL_CHUNKS = N_DEPTHS * N_CHUNKS


def _make_sc_gather():
  mesh = plsc.VectorSubcoreMesh(core_axis_name="c", subcore_axis_name="s")

  @functools.partial(
      pl.kernel,
      mesh=mesh,
      out_type=[jax.ShapeDtypeStruct((BATCH, D_PAD), jnp.int32)
                for _ in range(N_DEPTHS)],
      scratch_types=[
          pltpu.VMEM((B_PER_W,), jnp.int32),
          pltpu.VMEM((2, CHUNK), jnp.int32),
          pltpu.VMEM((2, CHUNK, D_PAD), jnp.int32),
          pltpu.SemaphoreType.DMA,
          pltpu.SemaphoreType.DMA,
      ],
  )
  def gather_kernel(table, labels, out0, out1, out2, out3,
                    lab_v, idx_v, rows_v, sem0, sem1):
    outs = (out0, out1, out2, out3)
    sems = (sem0, sem1)
    wid = lax.axis_index("s") * NUM_CORES + lax.axis_index("c")
    base = wid * B_PER_W

    pltpu.sync_copy(labels.at[pl.ds(base, B_PER_W)], lab_v)

    def fill_idx(g):
      d, c = divmod(g, N_CHUNKS)
      buf = g % 2
      off = jnp.int32(d * N_LABELS)
      for i in range(CHUNK // 16):
        sl = pl.ds(c * CHUNK + i * 16, 16)
        idx_v[buf, pl.ds(i * 16, 16)] = lab_v[sl] + off

    def start_gather(g):
      buf = g % 2
      copy = pltpu.make_async_copy(
          table.at[idx_v.at[buf]], rows_v.at[buf], sems[buf])
      copy.start()
      return copy

    def drain(g, copy):
      d, c = divmod(g, N_CHUNKS)
      buf = g % 2
      copy.wait()
      pltpu.sync_copy(rows_v.at[buf],
                      outs[d].at[pl.ds(base + c * CHUNK, CHUNK)])

    fill_idx(0)
    inflight = start_gather(0)
    for g in range(1, TOTAL_CHUNKS):
      fill_idx(g)
      nxt = start_gather(g)
      drain(g - 1, inflight)
      inflight = nxt
    drain(TOTAL_CHUNKS - 1, inflight)

  return gather_kernel


_sc_gather = _make_sc_gather()


@jax.jit
def kernel(labels, adversaries):
  tbl_u8 = adversaries.astype(jnp.uint8).reshape(
      N_DEPTHS * N_LABELS, D_WORDS, 4)
  tbl_i32 = jax.lax.bitcast_convert_type(tbl_u8, jnp.int32)
  tbl_i32 = jnp.pad(tbl_i32, ((0, 0), (0, D_PAD - D_WORDS)))
  lab = labels.reshape(BATCH)

  outs = _sc_gather(tbl_i32, lab)

  def unpack(o):
    bytes_ = jax.lax.bitcast_convert_type(o[:, :D_WORDS], jnp.uint8)
    return bytes_.reshape(BATCH, N_LABELS).astype(jnp.bool_)

  return tuple(unpack(o) for o in outs)


# speedup vs baseline: 2.7000x; 2.7000x over previous
import functools

import jax
import jax.numpy as jnp
from jax import lax
from jax.experimental import pallas as pl
from jax.experimental.pallas import tpu as pltpu
from jax.experimental.pallas import tpu_sc as plsc

N_LABELS = 1000
N_DEPTHS = 4
BATCH = 16384
D_PAD = 1024
D_WORDS = D_PAD // 4
PLANE = D_PAD // 4

NUM_CORES = 2
NUM_SUBCORES = 16
NUM_WORKERS = NUM_CORES * NUM_SUBCORES
B_PER_W = BATCH // NUM_WORKERS
CHUNK = 128
N_CHUNKS = B_PER_W // CHUNK
TOTAL_CHUNKS = N_DEPTHS * N_CHUNKS


def _make_sc_gather():
  mesh = plsc.VectorSubcoreMesh(core_axis_name="c", subcore_axis_name="s")

  @functools.partial(
      pl.kernel,
      mesh=mesh,
      out_type=[jax.ShapeDtypeStruct((BATCH, D_WORDS), jnp.int32)
                for _ in range(N_DEPTHS)],
      scratch_types=[
          pltpu.VMEM((B_PER_W,), jnp.int32),
          pltpu.VMEM((2, CHUNK), jnp.int32),
          pltpu.VMEM((2, CHUNK, D_WORDS), jnp.int32),
          pltpu.SemaphoreType.DMA,
          pltpu.SemaphoreType.DMA,
      ],
  )
  def gather_kernel(table, labels, out0, out1, out2, out3,
                    lab_v, idx_v, rows_v, sem0, sem1):
    outs = (out0, out1, out2, out3)
    sems = (sem0, sem1)
    wid = lax.axis_index("s") * NUM_CORES + lax.axis_index("c")
    base = wid * B_PER_W

    pltpu.sync_copy(labels.at[pl.ds(base, B_PER_W)], lab_v)

    def fill_idx(g):
      d, c = divmod(g, N_CHUNKS)
      buf = g % 2
      off = jnp.int32(d * N_LABELS)
      for i in range(CHUNK // 16):
        sl = pl.ds(c * CHUNK + i * 16, 16)
        idx_v[buf, pl.ds(i * 16, 16)] = lab_v[sl] + off

    def start_gather(g):
      buf = g % 2
      copy = pltpu.make_async_copy(
          table.at[idx_v.at[buf]], rows_v.at[buf], sems[buf])
      copy.start()
      return copy

    def drain(g, copy):
      d, c = divmod(g, N_CHUNKS)
      buf = g % 2
      copy.wait()
      pltpu.sync_copy(rows_v.at[buf],
                      outs[d].at[pl.ds(base + c * CHUNK, CHUNK)])

    fill_idx(0)
    inflight = start_gather(0)
    for g in range(1, TOTAL_CHUNKS):
      fill_idx(g)
      nxt = start_gather(g)
      drain(g - 1, inflight)
      inflight = nxt
    drain(TOTAL_CHUNKS - 1, inflight)

  return gather_kernel


_sc_gather = _make_sc_gather()


@jax.jit
def kernel(labels, adversaries):
  a = jnp.pad(adversaries, ((0, 0), (0, 0), (0, D_PAD - N_LABELS)))
  a = a.reshape(N_DEPTHS, N_LABELS, 4, PLANE).astype(jnp.uint32)
  w = (a[:, :, 0, :] | (a[:, :, 1, :] << 8)
       | (a[:, :, 2, :] << 16) | (a[:, :, 3, :] << 24))
  tbl = jax.lax.bitcast_convert_type(
      w.reshape(N_DEPTHS * N_LABELS, D_WORDS), jnp.int32)
  lab = labels.reshape(BATCH)

  outs = _sc_gather(tbl, lab)

  def unpack(o):
    ow = jax.lax.bitcast_convert_type(o, jnp.uint32)
    planes = [((ow >> (8 * m)) & 1) != 0 for m in range(4)]
    return jnp.concatenate(planes, axis=1)[:, :N_LABELS]

  return tuple(unpack(o) for o in outs)
